# baseline (speedup 1.0000x reference)
import functools

import jax
import jax.numpy as jnp
from jax import lax
from jax.experimental import pallas as pl
from jax.experimental.pallas import tpu as pltpu
from jax.experimental.pallas import tpu_sc as plsc

NUM_WORKERS = 32
P_CH = 4


def _sc_gather_add_t(table, x, pos):
    bsz, maxlen = x.shape
    d = table.shape[1]
    bb = bsz // NUM_WORKERS
    n_chunks = maxlen // P_CH
    ch_rows = P_CH * bb
    seg = d // 8
    mesh = plsc.VectorSubcoreMesh(core_axis_name="c", subcore_axis_name="s")

    @functools.partial(
        pl.kernel,
        mesh=mesh,
        out_type=jax.ShapeDtypeStruct((maxlen * d * bsz // 128, 128),
                                      jnp.float32),
        compiler_params=pltpu.CompilerParams(use_tc_tiling_on_sc=False,
                                             needs_layout_passes=False),
        scratch_types=[
            pltpu.VMEM((bb, maxlen), jnp.int32),
            pltpu.VMEM((ch_rows,), jnp.int32),
            pltpu.VMEM((ch_rows,), jnp.int32),
            pltpu.VMEM((ch_rows, d), jnp.float32),
            pltpu.VMEM((ch_rows, d), jnp.float32),
            pltpu.VMEM((bb, 128), jnp.float32),
            pltpu.VMEM((bb, 128), jnp.float32),
            pltpu.VMEM((maxlen, d), jnp.float32),
            pltpu.SemaphoreType.DMA,
            pltpu.SemaphoreType.DMA,
            pltpu.SemaphoreType.DMA,
            pltpu.SemaphoreType.DMA,
        ],
    )
    def gather_kernel(table_hbm, x_hbm, pos_hbm, out_hbm,
                      xblk, idx0, idx1, g0v, g1v, o0v, o1v, pos_v,
                      g0, g1, w0, w1):
        wid = lax.axis_index("s") * 2 + lax.axis_index("c")
        idx_v = (idx0, idx1)
        g_v = (g0v, g1v)
        o_v = (o0v, o1v)
        gsem = (g0, g1)
        wsem = (w0, w1)

        pltpu.sync_copy(x_hbm.at[pl.ds(wid * bb, bb)], xblk)
        pltpu.sync_copy(pos_hbm, pos_v)

        iota = lax.iota(jnp.int32, 16)
        rowv = [[jnp.full((16,), p_l * d + 16 * h, jnp.int32) + iota
                 for h in range(d // 16)] for p_l in range(P_CH)]

        def extract_idx(ci, b):
            for p_l in range(P_CH):
                col = jnp.full((16,), P_CH * ci + p_l, jnp.int32)
                for m in range(bb // 16):
                    rows = jnp.full((16,), 16 * m, jnp.int32) + iota
                    v = plsc.load_gather(xblk, [rows, col])
                    idx_v[b][pl.ds(p_l * bb + 16 * m, 16)] = v

        def start_gather(b):
            pltpu.async_copy(table_hbm.at[idx_v[b]], g_v[b], gsem[b])

        def wait_gather(b):
            pltpu.make_async_copy(table_hbm.at[idx_v[b]], g_v[b],
                                  gsem[b]).wait()

        def compute_scatter(ci, b):
            gv, ov = g_v[b], o_v[b]
            pvs = [[pos_v[P_CH * ci + p_l, pl.ds(16 * h, 16)]
                    for h in range(d // 16)] for p_l in range(P_CH)]

            @pl.loop(0, bb)
            def _(r):
                colv = jnp.full((16,), 0, jnp.int32) + r
                for p_l in range(P_CH):
                    for h in range(d // 16):
                        val = gv[p_l * bb + r, pl.ds(16 * h, 16)]
                        plsc.store_scatter(ov, [rowv[p_l][h], colv],
                                           val + pvs[p_l][h])

        def start_writeback(ci, b):
            for p_l in range(P_CH):
                p = P_CH * ci + p_l
                for e_blk in range(seg):
                    pltpu.async_copy(
                        o_v[b].at[pl.ds((p_l * seg + e_blk) * 8, 8)],
                        out_hbm.at[pl.ds(p * (seg * bsz // 16)
                                         + e_blk * (bsz // 16) + wid * 8, 8)],
                        wsem[b])

        def wait_writeback(ci, b):
            for p_l in range(P_CH):
                p = P_CH * ci + p_l
                for e_blk in range(seg):
                    pltpu.make_async_copy(
                        o_v[b].at[pl.ds((p_l * seg + e_blk) * 8, 8)],
                        out_hbm.at[pl.ds(p * (seg * bsz // 16)
                                         + e_blk * (bsz // 16) + wid * 8, 8)],
                        wsem[b]).wait()

        extract_idx(0, 0)
        start_gather(0)

        @pl.loop(0, n_chunks, step=2)
        def _(ci):
            for b in range(2):
                cur = ci + b
                nxt = cur + 1

                @pl.when(nxt < n_chunks)
                def _():
                    extract_idx(nxt, 1 - b)
                    start_gather(1 - b)

                wait_gather(b)

                @pl.when(cur >= 2)
                def _():
                    wait_writeback(cur - 2, b)

                compute_scatter(cur, b)
                start_writeback(cur, b)

        wait_writeback(n_chunks - 2, 0)
        wait_writeback(n_chunks - 1, 1)

    return gather_kernel(table, x, pos)


def kernel(x, token_table, pos_table):
    b, maxlen = x.shape
    d = token_table.shape[1]
    out128 = _sc_gather_add_t(token_table, x.astype(jnp.int32), pos_table)
    return (out128.reshape(maxlen, d // 8, b // 128, 8, 128)
            .transpose(2, 4, 0, 1, 3)
            .reshape(b, maxlen, d))


# speedup vs baseline: 1.3970x; 1.3970x over previous
import functools

import jax
import jax.numpy as jnp
from jax import lax
from jax.experimental import pallas as pl
from jax.experimental.pallas import tpu as pltpu
from jax.experimental.pallas import tpu_sc as plsc

NUM_WORKERS = 32
P_CH = 4


def _sc_gather_add_t(table, x, pos):
    bsz, maxlen = x.shape
    d = table.shape[1]
    bb = bsz // NUM_WORKERS
    n_chunks = maxlen // P_CH
    ch_rows = P_CH * bb
    seg = d // 8
    mesh = plsc.VectorSubcoreMesh(core_axis_name="c", subcore_axis_name="s")

    @functools.partial(
        pl.kernel,
        mesh=mesh,
        out_type=jax.ShapeDtypeStruct((maxlen * d * bsz // 128, 128),
                                      jnp.float32),
        compiler_params=pltpu.CompilerParams(use_tc_tiling_on_sc=False,
                                             needs_layout_passes=False),
        scratch_types=[
            pltpu.VMEM((bb, maxlen), jnp.int32),
            pltpu.VMEM((ch_rows,), jnp.int32),
            pltpu.VMEM((ch_rows,), jnp.int32),
            pltpu.VMEM((ch_rows, d), jnp.float32),
            pltpu.VMEM((ch_rows, d), jnp.float32),
            pltpu.VMEM((bb, 128), jnp.float32),
            pltpu.VMEM((bb, 128), jnp.float32),
            pltpu.VMEM((maxlen, d), jnp.float32),
            pltpu.SemaphoreType.DMA,
            pltpu.SemaphoreType.DMA,
            pltpu.SemaphoreType.DMA,
            pltpu.SemaphoreType.DMA,
        ],
    )
    def gather_kernel(table_hbm, x_hbm, pos_hbm, out_hbm,
                      xblk, idx0, idx1, g0v, g1v, o0v, o1v, pos_v,
                      g0, g1, w0, w1):
        wid = lax.axis_index("s") * 2 + lax.axis_index("c")
        idx_v = (idx0, idx1)
        g_v = (g0v, g1v)
        o_v = (o0v, o1v)
        gsem = (g0, g1)
        wsem = (w0, w1)

        pltpu.sync_copy(x_hbm.at[pl.ds(wid * bb, bb)], xblk)
        pltpu.sync_copy(pos_hbm, pos_v)

        iota = lax.iota(jnp.int32, 16)
        rowv = [[jnp.full((16,), p_l * d + 16 * h, jnp.int32) + iota
                 for h in range(d // 16)] for p_l in range(P_CH)]

        def extract_idx(ci, b):
            for p_l in range(P_CH):
                col = jnp.full((16,), P_CH * ci + p_l, jnp.int32)
                for m in range(bb // 16):
                    rows = jnp.full((16,), 16 * m, jnp.int32) + iota
                    v = plsc.load_gather(xblk, [rows, col])
                    idx_v[b][pl.ds(p_l * bb + 16 * m, 16)] = v

        def start_gather(b):
            pltpu.async_copy(table_hbm.at[idx_v[b]], g_v[b], gsem[b])

        def wait_gather(b):
            pltpu.make_async_copy(table_hbm.at[idx_v[b]], g_v[b],
                                  gsem[b]).wait()

        def compute_scatter(ci, b):
            gv, ov = g_v[b], o_v[b]
            pvs = [[pos_v[P_CH * ci + p_l, pl.ds(16 * h, 16)]
                    for h in range(d // 16)] for p_l in range(P_CH)]

            @plsc.parallel_loop(0, bb, 1, unroll=8)
            def _(r):
                colv = jnp.full((16,), 0, jnp.int32) + r
                for p_l in range(P_CH):
                    for h in range(d // 16):
                        val = gv[p_l * bb + r, pl.ds(16 * h, 16)]
                        plsc.store_scatter(ov, [rowv[p_l][h], colv],
                                           val + pvs[p_l][h])

        def start_writeback(ci, b):
            for p_l in range(P_CH):
                p = P_CH * ci + p_l
                for e_blk in range(seg):
                    pltpu.async_copy(
                        o_v[b].at[pl.ds((p_l * seg + e_blk) * 8, 8)],
                        out_hbm.at[pl.ds(p * (seg * bsz // 16)
                                         + e_blk * (bsz // 16) + wid * 8, 8)],
                        wsem[b])

        def wait_writeback(ci, b):
            for p_l in range(P_CH):
                p = P_CH * ci + p_l
                for e_blk in range(seg):
                    pltpu.make_async_copy(
                        o_v[b].at[pl.ds((p_l * seg + e_blk) * 8, 8)],
                        out_hbm.at[pl.ds(p * (seg * bsz // 16)
                                         + e_blk * (bsz // 16) + wid * 8, 8)],
                        wsem[b]).wait()

        extract_idx(0, 0)
        start_gather(0)

        @pl.loop(0, n_chunks, step=2)
        def _(ci):
            for b in range(2):
                cur = ci + b
                nxt = cur + 1

                @pl.when(nxt < n_chunks)
                def _():
                    extract_idx(nxt, 1 - b)
                    start_gather(1 - b)

                wait_gather(b)

                @pl.when(cur >= 2)
                def _():
                    wait_writeback(cur - 2, b)

                compute_scatter(cur, b)
                start_writeback(cur, b)

        wait_writeback(n_chunks - 2, 0)
        wait_writeback(n_chunks - 1, 1)

    return gather_kernel(table, x, pos)


def kernel(x, token_table, pos_table):
    b, maxlen = x.shape
    d = token_table.shape[1]
    out128 = _sc_gather_add_t(token_table, x.astype(jnp.int32), pos_table)
    return (out128.reshape(maxlen, d // 8, b // 128, 8, 128)
            .transpose(2, 4, 0, 1, 3)
            .reshape(b, maxlen, d))


# speedup vs baseline: 2.1655x; 1.5501x over previous
import functools

import jax
import jax.numpy as jnp
from jax import lax
from jax.experimental import pallas as pl
from jax.experimental.pallas import tpu as pltpu
from jax.experimental.pallas import tpu_sc as plsc

NUM_WORKERS = 32
CHUNK = 1600


def _sc_gather_add(table, idx, pos):
    maxlen = pos.shape[0]
    d = table.shape[1]
    n = idx.shape[0] * idx.shape[1]
    reps = CHUNK // maxlen
    per_w = n // NUM_WORKERS
    n_chunks = per_w // CHUNK
    mesh = plsc.VectorSubcoreMesh(core_axis_name="c", subcore_axis_name="s")

    @functools.partial(
        pl.kernel,
        mesh=mesh,
        out_type=jax.ShapeDtypeStruct((n, 128), jnp.float32),
        compiler_params=pltpu.CompilerParams(use_tc_tiling_on_sc=False),
        scratch_types=[
            pltpu.VMEM((reps, maxlen), jnp.int32),
            pltpu.VMEM((reps, maxlen), jnp.int32),
            pltpu.VMEM((CHUNK, d), jnp.float32),
            pltpu.VMEM((CHUNK, d), jnp.float32),
            pltpu.VMEM((maxlen, d), jnp.float32),
            pltpu.SemaphoreType.DMA,
            pltpu.SemaphoreType.DMA,
            pltpu.SemaphoreType.DMA,
            pltpu.SemaphoreType.DMA,
        ],
    )
    def gather_kernel(table_hbm, idx_hbm, pos_hbm, out_hbm,
                      idx0, idx1, rows0, rows1, pos_v, g0, g1, w0, w1):
        wid = lax.axis_index("s") * 2 + lax.axis_index("c")
        base = wid * per_w
        idx_v = (idx0, idx1)
        rows_v = (rows0, rows1)
        gsem = (g0, g1)
        wsem = (w0, w1)

        pltpu.sync_copy(pos_hbm, pos_v)

        def add_pos(b):
            rows = rows_v[b]

            @pl.loop(0, maxlen)
            def _(p):
                for h in range(d // 16):
                    pv = pos_v[p, pl.ds(h * 16, 16)]
                    for t in range(reps):
                        plsc.addupdate(
                            rows.at[t * maxlen + p, pl.ds(h * 16, 16)], pv)

        def start_gather(ci, b):
            row0 = (base + ci * CHUNK) // maxlen
            pltpu.sync_copy(idx_hbm.at[pl.ds(row0, reps)], idx_v[b])
            for k in range(reps):
                pltpu.async_copy(table_hbm.at[idx_v[b].at[k]],
                                 rows_v[b].at[pl.ds(k * maxlen, maxlen)],
                                 gsem[b])

        def wait_gather(b):
            for k in range(reps):
                pltpu.make_async_copy(table_hbm.at[idx_v[b].at[k]],
                                      rows_v[b].at[pl.ds(k * maxlen, maxlen)],
                                      gsem[b]).wait()

        def start_writeback(ci, b):
            off = base + ci * CHUNK
            pltpu.async_copy(
                rows_v[b],
                out_hbm.at[pl.ds(off, CHUNK), pl.ds(0, d)],
                wsem[b])

        def wait_writeback(ci, b):
            off = base + ci * CHUNK
            pltpu.make_async_copy(
                rows_v[b],
                out_hbm.at[pl.ds(off, CHUNK), pl.ds(0, d)],
                wsem[b]).wait()

        start_gather(0, 0)

        @pl.loop(0, n_chunks, step=2)
        def _(ci):
            for b in range(2):
                cur = ci + b
                nxt = cur + 1

                @pl.when(nxt < n_chunks)
                def _():
                    @pl.when(nxt >= 2)
                    def _():
                        wait_writeback(nxt - 2, 1 - b)
                    start_gather(nxt, 1 - b)

                wait_gather(b)
                add_pos(b)
                start_writeback(cur, b)

        wait_writeback(n_chunks - 2, 0)
        wait_writeback(n_chunks - 1, 1)

    return gather_kernel(table, idx, pos)


def kernel(x, token_table, pos_table):
    b, maxlen = x.shape
    d = token_table.shape[1]
    out128 = _sc_gather_add(token_table, x.astype(jnp.int32), pos_table)
    return out128[:, :d].reshape(b, maxlen, d)
